# baseline (speedup 1.0000x reference)
import functools

import jax
import jax.numpy as jnp
from jax import lax
from jax.experimental import pallas as pl
from jax.experimental.pallas import tpu as pltpu
from jax.experimental.pallas import tpu_sc as plsc

_CONF = 0.3
_IOU = 0.6
_MAXWH = 4096.0
_MAXDET = 300

_NCLS = 80
_NPAD = 5120
_SUB, _LANE = 8, 640
_NG = _NPAD // 16
_CAP = _NPAD + 16
_LCAP = 304
_NEG = -3.0e38

_R_SCO, _R_CLS = 0, 1
_R_SX1, _R_SY1, _R_SX2, _R_SY2, _R_AR = 2, 3, 4, 5, 6
_R_X1, _R_Y1, _R_X2, _R_Y2 = 7, 8, 9, 10


def _prep_body(pt_ref, out_ref):
    obj = pt_ref[4]
    m = pt_ref[5] * obj
    j = jnp.zeros((_SUB, _LANE), jnp.float32)
    for c in range(1, _NCLS):
        v = pt_ref[5 + c] * obj
        j = jnp.where(v > m, jnp.float32(c), j)
        m = jnp.maximum(m, v)
    cx, cy = pt_ref[0], pt_ref[1]
    w, h = pt_ref[2], pt_ref[3]
    x1 = cx - w / 2.0
    y1 = cy - h / 2.0
    x2 = cx + w / 2.0
    y2 = cy + h / 2.0
    off = j * _MAXWH
    sx1 = x1 + off
    sy1 = y1 + off
    sx2 = x2 + off
    sy2 = y2 + off
    area = (sx2 - sx1) * (sy2 - sy1)
    valid = (obj > _CONF) & (m > _CONF)
    out_ref[_R_SCO] = jnp.where(valid, m, -1.0)
    out_ref[_R_CLS] = j
    out_ref[_R_SX1] = sx1
    out_ref[_R_SY1] = sy1
    out_ref[_R_SX2] = sx2
    out_ref[_R_SY2] = sy2
    out_ref[_R_AR] = area
    out_ref[_R_X1] = x1
    out_ref[_R_Y1] = y1
    out_ref[_R_X2] = x2
    out_ref[_R_Y2] = y2


def _lanes():
    return lax.iota(jnp.int32, 16)


def _argmax_groups(ref, ngrp):
    lanes = _lanes()

    def body(g, carry):
        mx, p = carry
        v = ref[pl.ds(g * 16, 16)]
        gm = jnp.max(v)
        gp = jnp.min(jnp.where(v == gm, g * 16 + lanes, jnp.int32(2 ** 30)))
        better = gm > mx
        return jnp.where(better, gm, mx), jnp.where(better, gp, p)

    return lax.fori_loop(0, ngrp, body, (jnp.float32(_NEG), jnp.int32(0)))


def _sc_store1(ref, pos, val):
    plsc.store_scatter(ref, [jnp.full((16,), pos, jnp.int32)],
                       jnp.full((16,), val), mask=_lanes() == 0)


def _read1(ref, pos):
    return ref[pl.ds(pos, 16)][0]


def _nms_stage1(nw, nc):
    def body(prep_hbm, ksco_hbm, kidx_hbm,
             sco_v, cls_v, sx1_v, sy1_v, sx2_v, sy2_v, ar_v,
             cidx_v, csco_v, cx1_v, cy1_v, cx2_v, cy2_v, car_v,
             ksco_v, kidx_v, dsem):
        w = lax.axis_index("s") * nc + lax.axis_index("c")
        lanes = _lanes()

        cps = [pltpu.async_copy(prep_hbm.at[r], v, dsem)
               for r, v in ((_R_SCO, sco_v), (_R_CLS, cls_v),
                            (_R_SX1, sx1_v), (_R_SY1, sy1_v),
                            (_R_SX2, sx2_v), (_R_SY2, sy2_v),
                            (_R_AR, ar_v))]
        for cp in cps:
            cp.wait()

        base, rem = _NCLS // nw, _NCLS % nw
        lo = jnp.where(w < rem, w * (base + 1),
                       rem * (base + 1) + (w - rem) * base)
        ncls_w = jnp.where(w < rem, base + 1, base)
        flo = lo.astype(jnp.float32)
        fhi = (lo + ncls_w - 1).astype(jnp.float32)

        def scan_body(h, off):
            for u in range(4):
                g = h * 4 + u
                cv = cls_v[pl.ds(g * 16, 16)]
                sv = sco_v[pl.ds(g * 16, 16)]
                mm = (cv >= flo) & (cv <= fhi) & (sv > 0.0)
                plsc.store_compressed(cidx_v.at[pl.ds(off, 16)],
                                      g * 16 + lanes, mask=mm)
                plsc.store_compressed(csco_v.at[pl.ds(off, 16)], sv, mask=mm)
                off = off + jnp.sum(mm.astype(jnp.int32))
            return off

        ncand = lax.fori_loop(0, _NG // 4, scan_body, jnp.int32(0))

        cidx_v[pl.ds(ncand, 16)] = jnp.zeros((16,), jnp.int32)
        csco_v[pl.ds(ncand, 16)] = jnp.full((16,), _NEG, jnp.float32)
        ngrp = (ncand + 15) // 16

        def gather_body(g, _):
            iv = cidx_v[pl.ds(g * 16, 16)]
            cx1_v[pl.ds(g * 16, 16)] = plsc.load_gather(sx1_v, [iv])
            cy1_v[pl.ds(g * 16, 16)] = plsc.load_gather(sy1_v, [iv])
            cx2_v[pl.ds(g * 16, 16)] = plsc.load_gather(sx2_v, [iv])
            cy2_v[pl.ds(g * 16, 16)] = plsc.load_gather(sy2_v, [iv])
            car_v[pl.ds(g * 16, 16)] = plsc.load_gather(ar_v, [iv])
            return 0

        lax.fori_loop(0, ngrp, gather_body, 0)

        def cond(carry):
            k, mx, _ = carry
            return (k < _MAXDET) & (mx > 0.0)

        def gbody(carry):
            k, mx, pos = carry
            gidx = _read1(cidx_v, pos)
            bx1 = _read1(cx1_v, pos)
            by1 = _read1(cy1_v, pos)
            bx2 = _read1(cx2_v, pos)
            by2 = _read1(cy2_v, pos)
            bar = _read1(car_v, pos)
            _sc_store1(kidx_v, k, gidx)
            _sc_store1(ksco_v, k, mx)

            def supp(g, carry):
                cm, cp = carry
                sl = pl.ds(g * 16, 16)
                ix = jnp.maximum(
                    jnp.minimum(cx2_v[sl], bx2) - jnp.maximum(cx1_v[sl], bx1),
                    0.0)
                iy = jnp.maximum(
                    jnp.minimum(cy2_v[sl], by2) - jnp.maximum(cy1_v[sl], by1),
                    0.0)
                inter = ix * iy
                iou = inter / (car_v[sl] + bar - inter + 1e-9)
                kill = (iou > _IOU) | ((g * 16 + lanes) == pos)
                ns = jnp.where(kill, _NEG, csco_v[sl])
                csco_v[sl] = ns
                gm = jnp.max(ns)
                gp = jnp.min(jnp.where(ns == gm, g * 16 + lanes,
                                       jnp.int32(2 ** 30)))
                better = gm > cm
                return (jnp.where(better, gm, cm),
                        jnp.where(better, gp, cp))

            m2, p2 = lax.fori_loop(0, ngrp, supp,
                                   (jnp.float32(_NEG), jnp.int32(0)))
            return k + 1, m2, p2

        m0, p0 = _argmax_groups(csco_v, ngrp)
        kfin, _, _ = lax.while_loop(cond, gbody, (jnp.int32(0), m0, p0))

        _sc_store1(kidx_v, _MAXDET, kfin)
        pltpu.sync_copy(ksco_v, ksco_hbm.at[w])
        pltpu.sync_copy(kidx_v, kidx_hbm.at[w])

    return body


def _nms_stage2(nw, nc):
    def body(prep_hbm, ksco_hbm, kidx_hbm, out_hbm,
             sco_l, idx_l, x1_v, y1_v, x2_v, y2_v, cls_v,
             mg_idx, mg_sco, outv, dsem):
        w = lax.axis_index("s") * nc + lax.axis_index("c")
        lanes = _lanes()
        hgrp = nw // 16

        @pl.when(w == 0)
        def _():
            cps = [pltpu.async_copy(src, dst, dsem)
                   for src, dst in ((ksco_hbm, sco_l), (kidx_hbm, idx_l),
                                    (prep_hbm.at[_R_X1], x1_v),
                                    (prep_hbm.at[_R_Y1], y1_v),
                                    (prep_hbm.at[_R_X2], x2_v),
                                    (prep_hbm.at[_R_Y2], y2_v),
                                    (prep_hbm.at[_R_CLS], cls_v))]
            for cp in cps:
                cp.wait()

            def mg_init(g, _):
                mg_idx[pl.ds(g * 16, 16)] = jnp.zeros((16,), jnp.int32)
                return 0

            lax.fori_loop(0, _LCAP // 16, mg_init, 0)

            h, p, cn = [], [], []
            for t in range(hgrp):
                c16 = t * 16 + lanes
                cnt16 = plsc.load_gather(
                    idx_l, [c16, jnp.full((16,), _MAXDET, jnp.int32)])
                h16 = plsc.load_gather(
                    sco_l, [c16, jnp.zeros((16,), jnp.int32)])
                h.append(jnp.where(cnt16 > 0, h16, _NEG))
                p.append(jnp.zeros((16,), jnp.int32))
                cn.append(cnt16)

            def find(hs):
                mx = jnp.max(hs[0])
                grp = jnp.int32(0)
                for t in range(1, hgrp):
                    mt = jnp.max(hs[t])
                    better = mt > mx
                    mx = jnp.where(better, mt, mx)
                    grp = jnp.where(better, jnp.int32(t), grp)
                hsel = hs[0]
                for t in range(1, hgrp):
                    hsel = jnp.where(grp == t, hs[t], hsel)
                lane = jnp.min(jnp.where(hsel == mx, lanes, jnp.int32(16)))
                return mx, grp, lane

            def cond(carry):
                r, mx = carry[0], carry[1]
                return (r < _MAXDET) & (mx > 0.0)

            def mbody(carry):
                r, mx, grp, lane = carry[:4]
                hs = list(carry[4:4 + hgrp])
                ps = list(carry[4 + hgrp:4 + 2 * hgrp])
                lmask = lanes == lane
                psel = ps[0]
                csel = cn[0]
                for t in range(1, hgrp):
                    psel = jnp.where(grp == t, ps[t], psel)
                    csel = jnp.where(grp == t, cn[t], csel)
                hp = jnp.sum(jnp.where(lmask, psel, 0))
                cnl = jnp.sum(jnp.where(lmask, csel, 0))
                cpos = grp * 16 + lane
                cvec = jnp.full((16,), cpos, jnp.int32)
                gidx = plsc.load_gather(
                    idx_l, [cvec, jnp.full((16,), hp, jnp.int32)])[0]
                _sc_store1(mg_idx, r, gidx)
                _sc_store1(mg_sco, r, mx)
                hp2 = hp + 1
                nh_raw = plsc.load_gather(
                    sco_l,
                    [cvec, jnp.full((16,), jnp.minimum(hp2, _MAXDET - 1),
                                    jnp.int32)])[0]
                nhead = jnp.where(hp2 < cnl, nh_raw, jnp.float32(_NEG))
                for t in range(hgrp):
                    sel = (grp == t) & lmask
                    hs[t] = jnp.where(sel, nhead, hs[t])
                    ps[t] = jnp.where(sel, hp2, ps[t])
                m2, g2, l2 = find(hs)
                return (r + 1, m2, g2, l2, *hs, *ps)

            m0, g0, l0 = find(h)
            fin = lax.while_loop(cond, mbody,
                                 (jnp.int32(0), m0, g0, l0, *h, *p))
            rfin = fin[0]

            def asm(g, _):
                rows = g * 16 + lanes
                ok = rows < rfin
                wr = rows < _MAXDET
                iv = jnp.where(ok, mg_idx[pl.ds(g * 16, 16)], 0)
                vx1 = jnp.where(ok, plsc.load_gather(x1_v, [iv]), 0.0)
                plsc.store_scatter(outv, [rows, jnp.full((16,), 0, jnp.int32)],
                                   vx1, mask=wr)
                vy1 = jnp.where(ok, plsc.load_gather(y1_v, [iv]), 0.0)
                plsc.store_scatter(outv, [rows, jnp.full((16,), 1, jnp.int32)],
                                   vy1, mask=wr)
                vx2 = jnp.where(ok, plsc.load_gather(x2_v, [iv]), 0.0)
                plsc.store_scatter(outv, [rows, jnp.full((16,), 2, jnp.int32)],
                                   vx2, mask=wr)
                vy2 = jnp.where(ok, plsc.load_gather(y2_v, [iv]), 0.0)
                plsc.store_scatter(outv, [rows, jnp.full((16,), 3, jnp.int32)],
                                   vy2, mask=wr)
                vcf = jnp.where(ok, mg_sco[pl.ds(g * 16, 16)], 0.0)
                plsc.store_scatter(outv, [rows, jnp.full((16,), 4, jnp.int32)],
                                   vcf, mask=wr)
                vcl = jnp.where(ok, plsc.load_gather(cls_v, [iv]), 0.0)
                plsc.store_scatter(outv, [rows, jnp.full((16,), 5, jnp.int32)],
                                   vcl, mask=wr)
                return 0

            lax.fori_loop(0, _LCAP // 16, asm, 0)
            pltpu.sync_copy(outv, out_hbm)

    return body


def kernel(x):
    p = x[0, 0]
    pt = jnp.pad(jnp.transpose(p), ((0, 0), (0, _NPAD - p.shape[0])))
    pt = pt.reshape(_NCLS + 5, _SUB, _LANE)
    prep = pl.pallas_call(
        _prep_body,
        out_shape=jax.ShapeDtypeStruct((16, _SUB, _LANE), jnp.float32),
    )(pt)
    prep = prep.reshape(16, _NPAD)

    info = plsc.get_sparse_core_info()
    nc, ns = info.num_cores, info.num_subcores
    nw = nc * ns
    mesh = plsc.VectorSubcoreMesh(core_axis_name="c", subcore_axis_name="s",
                                  num_cores=nc, num_subcores=ns)
    cp = pltpu.CompilerParams(needs_layout_passes=False)

    f32, i32 = jnp.float32, jnp.int32
    stage1 = functools.partial(
        pl.kernel, _nms_stage1(nw, nc), mesh=mesh, compiler_params=cp,
        out_type=(jax.ShapeDtypeStruct((nw, _LCAP), f32),
                  jax.ShapeDtypeStruct((nw, _LCAP), i32)),
        scratch_types=(
            [pltpu.VMEM((_NPAD,), f32)] * 7
            + [pltpu.VMEM((_CAP,), i32)]
            + [pltpu.VMEM((_CAP,), f32)] * 6
            + [pltpu.VMEM((_LCAP,), f32),
               pltpu.VMEM((_LCAP,), i32),
               pltpu.SemaphoreType.DMA]),
    )()
    ksco, kidx = stage1(prep)

    stage2 = functools.partial(
        pl.kernel, _nms_stage2(nw, nc), mesh=mesh, compiler_params=cp,
        out_type=jax.ShapeDtypeStruct((_MAXDET, 6), f32),
        scratch_types=(
            [pltpu.VMEM((nw, _LCAP), f32), pltpu.VMEM((nw, _LCAP), i32)]
            + [pltpu.VMEM((_NPAD,), f32)] * 5
            + [pltpu.VMEM((_LCAP + 16,), i32), pltpu.VMEM((_LCAP + 16,), f32),
               pltpu.VMEM((_MAXDET, 6), f32),
               pltpu.SemaphoreType.DMA]),
    )()
    return stage2(prep, ksco, kidx)


def build_and_compile():
    x = jax.ShapeDtypeStruct((1, 1, 5000, 85), jnp.float32)
    jax.jit(kernel).lower(x).compile()


# speedup vs baseline: 1.0225x; 1.0225x over previous
import functools

import jax
import jax.numpy as jnp
from jax import lax
from jax.experimental import pallas as pl
from jax.experimental.pallas import tpu as pltpu
from jax.experimental.pallas import tpu_sc as plsc

_CONF = 0.3
_IOU = 0.6
_MAXWH = 4096.0
_MAXDET = 300

_NCLS = 80
_NPAD = 5120
_SUB, _LANE = 8, 640
_NG = _NPAD // 16
_CAP = _NPAD + 16
_LCAP = 304
_NEG = -3.0e38

_R_SCO, _R_CLS = 0, 1
_R_SX1, _R_SY1, _R_SX2, _R_SY2, _R_AR = 2, 3, 4, 5, 6
_R_X1, _R_Y1, _R_X2, _R_Y2 = 7, 8, 9, 10


def _prep_body(pt_ref, out_ref):
    obj = pt_ref[4]
    m = pt_ref[5] * obj
    j = jnp.zeros((_SUB, _LANE), jnp.float32)
    for c in range(1, _NCLS):
        v = pt_ref[5 + c] * obj
        j = jnp.where(v > m, jnp.float32(c), j)
        m = jnp.maximum(m, v)
    cx, cy = pt_ref[0], pt_ref[1]
    w, h = pt_ref[2], pt_ref[3]
    x1 = cx - w / 2.0
    y1 = cy - h / 2.0
    x2 = cx + w / 2.0
    y2 = cy + h / 2.0
    off = j * _MAXWH
    sx1 = x1 + off
    sy1 = y1 + off
    sx2 = x2 + off
    sy2 = y2 + off
    area = (sx2 - sx1) * (sy2 - sy1)
    valid = (obj > _CONF) & (m > _CONF)
    out_ref[_R_SCO] = jnp.where(valid, m, -1.0)
    out_ref[_R_CLS] = j
    out_ref[_R_SX1] = sx1
    out_ref[_R_SY1] = sy1
    out_ref[_R_SX2] = sx2
    out_ref[_R_SY2] = sy2
    out_ref[_R_AR] = area
    out_ref[_R_X1] = x1
    out_ref[_R_Y1] = y1
    out_ref[_R_X2] = x2
    out_ref[_R_Y2] = y2


def _lanes():
    return lax.iota(jnp.int32, 16)


def _argmax_groups(ref, ngrp):
    lanes = _lanes()

    def body(g, carry):
        mx, p = carry
        v = ref[pl.ds(g * 16, 16)]
        gm = jnp.max(v)
        gp = jnp.min(jnp.where(v == gm, g * 16 + lanes, jnp.int32(2 ** 30)))
        better = gm > mx
        return jnp.where(better, gm, mx), jnp.where(better, gp, p)

    return lax.fori_loop(0, ngrp, body, (jnp.float32(_NEG), jnp.int32(0)))


def _sc_store1(ref, pos, val):
    plsc.store_scatter(ref, [jnp.full((16,), pos, jnp.int32)],
                       jnp.full((16,), val), mask=_lanes() == 0)


def _read1(ref, pos):
    return ref[pl.ds(pos, 16)][0]


def _nms_stage1(nw, nc):
    def body(prep_hbm, ksco_hbm, kidx_hbm,
             sco_v, cls_v, sx1_v, sy1_v, sx2_v, sy2_v, ar_v,
             cidx_v, csco_v, cx1_v, cy1_v, cx2_v, cy2_v, car_v,
             ksco_v, kidx_v, dsem):
        w = lax.axis_index("s") * nc + lax.axis_index("c")
        lanes = _lanes()

        cps = [pltpu.async_copy(prep_hbm.at[r], v, dsem)
               for r, v in ((_R_SCO, sco_v), (_R_CLS, cls_v),
                            (_R_SX1, sx1_v), (_R_SY1, sy1_v),
                            (_R_SX2, sx2_v), (_R_SY2, sy2_v),
                            (_R_AR, ar_v))]
        for cp in cps:
            cp.wait()

        base, rem = _NCLS // nw, _NCLS % nw
        lo = jnp.where(w < rem, w * (base + 1),
                       rem * (base + 1) + (w - rem) * base)
        ncls_w = jnp.where(w < rem, base + 1, base)
        flo = lo.astype(jnp.float32)
        fhi = (lo + ncls_w - 1).astype(jnp.float32)

        def scan_body(h, off):
            for u in range(4):
                g = h * 4 + u
                cv = cls_v[pl.ds(g * 16, 16)]
                sv = sco_v[pl.ds(g * 16, 16)]
                mm = (cv >= flo) & (cv <= fhi) & (sv > 0.0)
                plsc.store_compressed(cidx_v.at[pl.ds(off, 16)],
                                      g * 16 + lanes, mask=mm)
                plsc.store_compressed(csco_v.at[pl.ds(off, 16)], sv, mask=mm)
                off = off + jnp.sum(mm.astype(jnp.int32))
            return off

        ncand = lax.fori_loop(0, _NG // 4, scan_body, jnp.int32(0))

        cidx_v[pl.ds(ncand, 16)] = jnp.zeros((16,), jnp.int32)
        csco_v[pl.ds(ncand, 16)] = jnp.full((16,), _NEG, jnp.float32)
        ngrp = (ncand + 15) // 16

        def gather_body(g, _):
            iv = cidx_v[pl.ds(g * 16, 16)]
            cx1_v[pl.ds(g * 16, 16)] = plsc.load_gather(sx1_v, [iv])
            cy1_v[pl.ds(g * 16, 16)] = plsc.load_gather(sy1_v, [iv])
            cx2_v[pl.ds(g * 16, 16)] = plsc.load_gather(sx2_v, [iv])
            cy2_v[pl.ds(g * 16, 16)] = plsc.load_gather(sy2_v, [iv])
            car_v[pl.ds(g * 16, 16)] = plsc.load_gather(ar_v, [iv])
            return 0

        lax.fori_loop(0, ngrp, gather_body, 0)

        def cond(carry):
            k, mx, _ = carry
            return (k < _MAXDET) & (mx > 0.0)

        def gbody(carry):
            k, mx, pos = carry
            gidx = _read1(cidx_v, pos)
            bx1 = _read1(cx1_v, pos)
            by1 = _read1(cy1_v, pos)
            bx2 = _read1(cx2_v, pos)
            by2 = _read1(cy2_v, pos)
            bar = _read1(car_v, pos)
            _sc_store1(kidx_v, k, gidx)
            _sc_store1(ksco_v, k, mx)

            def supp(g, carry):
                cm, cp = carry
                sl = pl.ds(g * 16, 16)
                ix = jnp.maximum(
                    jnp.minimum(cx2_v[sl], bx2) - jnp.maximum(cx1_v[sl], bx1),
                    0.0)
                iy = jnp.maximum(
                    jnp.minimum(cy2_v[sl], by2) - jnp.maximum(cy1_v[sl], by1),
                    0.0)
                inter = ix * iy
                iou = inter / (car_v[sl] + bar - inter + 1e-9)
                kill = (iou > _IOU) | ((g * 16 + lanes) == pos)
                ns = jnp.where(kill, _NEG, csco_v[sl])
                csco_v[sl] = ns
                gm = jnp.max(ns)
                gp = jnp.min(jnp.where(ns == gm, g * 16 + lanes,
                                       jnp.int32(2 ** 30)))
                better = gm > cm
                return (jnp.where(better, gm, cm),
                        jnp.where(better, gp, cp))

            m2, p2 = lax.fori_loop(0, ngrp, supp,
                                   (jnp.float32(_NEG), jnp.int32(0)))
            return k + 1, m2, p2

        m0, p0 = _argmax_groups(csco_v, ngrp)
        kfin, _, _ = lax.while_loop(cond, gbody, (jnp.int32(0), m0, p0))

        _sc_store1(kidx_v, _MAXDET, kfin)
        pltpu.sync_copy(ksco_v, ksco_hbm.at[w])
        pltpu.sync_copy(kidx_v, kidx_hbm.at[w])

    return body


def _nms_stage2(nw, nc):
    def body(prep_hbm, ksco_hbm, kidx_hbm, out_hbm,
             sco_l, idx_l, x1_v, y1_v, x2_v, y2_v, cls_v,
             mg_idx, mg_sco, outv, dsem):
        w = lax.axis_index("s") * nc + lax.axis_index("c")
        lanes = _lanes()
        hgrp = nw // 16

        @pl.when(w == 0)
        def _():
            cps = [pltpu.async_copy(src, dst, dsem)
                   for src, dst in ((ksco_hbm, sco_l), (kidx_hbm, idx_l),
                                    (prep_hbm.at[_R_X1], x1_v),
                                    (prep_hbm.at[_R_Y1], y1_v),
                                    (prep_hbm.at[_R_X2], x2_v),
                                    (prep_hbm.at[_R_Y2], y2_v),
                                    (prep_hbm.at[_R_CLS], cls_v))]
            for cp in cps:
                cp.wait()

            def mg_init(g, _):
                mg_idx[pl.ds(g * 16, 16)] = jnp.zeros((16,), jnp.int32)
                return 0

            lax.fori_loop(0, _LCAP // 16, mg_init, 0)

            h, p, cn = [], [], []
            for t in range(hgrp):
                c16 = t * 16 + lanes
                cnt16 = plsc.load_gather(
                    idx_l, [c16, jnp.full((16,), _MAXDET, jnp.int32)])
                h16 = plsc.load_gather(
                    sco_l, [c16, jnp.zeros((16,), jnp.int32)])
                h.append(jnp.where(cnt16 > 0, h16, _NEG))
                p.append(jnp.zeros((16,), jnp.int32))
                cn.append(cnt16)

            def find(hs):
                mx = jnp.max(hs[0])
                grp = jnp.int32(0)
                for t in range(1, hgrp):
                    mt = jnp.max(hs[t])
                    better = mt > mx
                    mx = jnp.where(better, mt, mx)
                    grp = jnp.where(better, jnp.int32(t), grp)
                hsel = hs[0]
                for t in range(1, hgrp):
                    hsel = jnp.where(grp == t, hs[t], hsel)
                lane = jnp.min(jnp.where(hsel == mx, lanes, jnp.int32(16)))
                return mx, grp, lane

            def cond(carry):
                r, mx = carry[0], carry[1]
                return (r < _MAXDET) & (mx > 0.0)

            def mbody(carry):
                r, mx, grp, lane = carry[:4]
                hs = list(carry[4:4 + hgrp])
                ps = list(carry[4 + hgrp:4 + 2 * hgrp])
                lmask = lanes == lane
                psel = ps[0]
                csel = cn[0]
                for t in range(1, hgrp):
                    psel = jnp.where(grp == t, ps[t], psel)
                    csel = jnp.where(grp == t, cn[t], csel)
                hp = jnp.sum(jnp.where(lmask, psel, 0))
                cnl = jnp.sum(jnp.where(lmask, csel, 0))
                cpos = grp * 16 + lane
                cvec = jnp.full((16,), cpos, jnp.int32)
                gidx = plsc.load_gather(
                    idx_l, [cvec, jnp.full((16,), hp, jnp.int32)])[0]
                _sc_store1(mg_idx, r, gidx)
                _sc_store1(mg_sco, r, mx)
                hp2 = hp + 1
                nh_raw = plsc.load_gather(
                    sco_l,
                    [cvec, jnp.full((16,), jnp.minimum(hp2, _MAXDET - 1),
                                    jnp.int32)])[0]
                nhead = jnp.where(hp2 < cnl, nh_raw, jnp.float32(_NEG))
                for t in range(hgrp):
                    sel = (grp == t) & lmask
                    hs[t] = jnp.where(sel, nhead, hs[t])
                    ps[t] = jnp.where(sel, hp2, ps[t])
                m2, g2, l2 = find(hs)
                return (r + 1, m2, g2, l2, *hs, *ps)

            m0, g0, l0 = find(h)
            fin = lax.while_loop(cond, mbody,
                                 (jnp.int32(0), m0, g0, l0, *h, *p))
            rfin = fin[0]

            def asm(g, _):
                rows = g * 16 + lanes
                ok = rows < rfin
                iv = jnp.where(ok, mg_idx[pl.ds(g * 16, 16)], 0)
                flat = rows * 6
                vx1 = jnp.where(ok, plsc.load_gather(x1_v, [iv]), 0.0)
                plsc.store_scatter(outv, [flat], vx1)
                vy1 = jnp.where(ok, plsc.load_gather(y1_v, [iv]), 0.0)
                plsc.store_scatter(outv, [flat + 1], vy1)
                vx2 = jnp.where(ok, plsc.load_gather(x2_v, [iv]), 0.0)
                plsc.store_scatter(outv, [flat + 2], vx2)
                vy2 = jnp.where(ok, plsc.load_gather(y2_v, [iv]), 0.0)
                plsc.store_scatter(outv, [flat + 3], vy2)
                vcf = jnp.where(ok, mg_sco[pl.ds(g * 16, 16)], 0.0)
                plsc.store_scatter(outv, [flat + 4], vcf)
                vcl = jnp.where(ok, plsc.load_gather(cls_v, [iv]), 0.0)
                plsc.store_scatter(outv, [flat + 5], vcl)
                return 0

            lax.fori_loop(0, _LCAP // 16, asm, 0)
            pltpu.sync_copy(outv, out_hbm)

    return body


def kernel(x):
    p = x[0, 0]
    pt = jnp.pad(jnp.transpose(p), ((0, 0), (0, _NPAD - p.shape[0])))
    pt = pt.reshape(_NCLS + 5, _SUB, _LANE)
    prep = pl.pallas_call(
        _prep_body,
        out_shape=jax.ShapeDtypeStruct((16, _SUB, _LANE), jnp.float32),
    )(pt)
    prep = prep.reshape(16, _NPAD)

    info = plsc.get_sparse_core_info()
    nc, ns = info.num_cores, info.num_subcores
    nw = nc * ns
    mesh = plsc.VectorSubcoreMesh(core_axis_name="c", subcore_axis_name="s",
                                  num_cores=nc, num_subcores=ns)
    cp = pltpu.CompilerParams(needs_layout_passes=False)

    f32, i32 = jnp.float32, jnp.int32
    stage1 = functools.partial(
        pl.kernel, _nms_stage1(nw, nc), mesh=mesh, compiler_params=cp,
        out_type=(jax.ShapeDtypeStruct((nw, _LCAP), f32),
                  jax.ShapeDtypeStruct((nw, _LCAP), i32)),
        scratch_types=(
            [pltpu.VMEM((_NPAD,), f32)] * 7
            + [pltpu.VMEM((_CAP,), i32)]
            + [pltpu.VMEM((_CAP,), f32)] * 6
            + [pltpu.VMEM((_LCAP,), f32),
               pltpu.VMEM((_LCAP,), i32),
               pltpu.SemaphoreType.DMA]),
    )()
    ksco, kidx = stage1(prep)

    stage2 = functools.partial(
        pl.kernel, _nms_stage2(nw, nc), mesh=mesh, compiler_params=cp,
        out_type=jax.ShapeDtypeStruct((6 * _LCAP,), f32),
        scratch_types=(
            [pltpu.VMEM((nw, _LCAP), f32), pltpu.VMEM((nw, _LCAP), i32)]
            + [pltpu.VMEM((_NPAD,), f32)] * 5
            + [pltpu.VMEM((_LCAP + 16,), i32), pltpu.VMEM((_LCAP + 16,), f32),
               pltpu.VMEM((6 * _LCAP,), f32),
               pltpu.SemaphoreType.DMA]),
    )()
    outflat = stage2(prep, ksco, kidx)
    return outflat[:_MAXDET * 6].reshape(_MAXDET, 6)


def build_and_compile():
    x = jax.ShapeDtypeStruct((1, 1, 5000, 85), jnp.float32)
    jax.jit(kernel).lower(x).compile()


# speedup vs baseline: 1.1251x; 1.1003x over previous
import functools

import jax
import jax.numpy as jnp
from jax import lax
from jax.experimental import pallas as pl
from jax.experimental.pallas import tpu as pltpu
from jax.experimental.pallas import tpu_sc as plsc

_CONF = 0.3
_IOU = 0.6
_MAXWH = 4096.0
_MAXDET = 300

_NCLS = 80
_NPAD = 5120
_SUB, _LANE = 8, 640
_NG = _NPAD // 16
_CAP = _NPAD + 16
_LCAP = 304
_NEG = -3.0e38

_R_SCO, _R_CLS = 0, 1
_R_SX1, _R_SY1, _R_SX2, _R_SY2, _R_AR = 2, 3, 4, 5, 6
_R_X1, _R_Y1, _R_X2, _R_Y2 = 7, 8, 9, 10


def _prep_body(pt_ref, out_ref):
    obj = pt_ref[4]
    m = pt_ref[5] * obj
    j = jnp.zeros((_SUB, _LANE), jnp.float32)
    for c in range(1, _NCLS):
        v = pt_ref[5 + c] * obj
        j = jnp.where(v > m, jnp.float32(c), j)
        m = jnp.maximum(m, v)
    cx, cy = pt_ref[0], pt_ref[1]
    w, h = pt_ref[2], pt_ref[3]
    x1 = cx - w / 2.0
    y1 = cy - h / 2.0
    x2 = cx + w / 2.0
    y2 = cy + h / 2.0
    off = j * _MAXWH
    sx1 = x1 + off
    sy1 = y1 + off
    sx2 = x2 + off
    sy2 = y2 + off
    area = (sx2 - sx1) * (sy2 - sy1)
    valid = (obj > _CONF) & (m > _CONF)
    out_ref[_R_SCO] = jnp.where(valid, m, -1.0)
    out_ref[_R_CLS] = j
    out_ref[_R_SX1] = sx1
    out_ref[_R_SY1] = sy1
    out_ref[_R_SX2] = sx2
    out_ref[_R_SY2] = sy2
    out_ref[_R_AR] = area
    out_ref[_R_X1] = x1
    out_ref[_R_Y1] = y1
    out_ref[_R_X2] = x2
    out_ref[_R_Y2] = y2


def _lanes():
    return lax.iota(jnp.int32, 16)


def _argmax_groups(ref, ngrp):
    lanes = _lanes()

    def body(g, carry):
        mx, p = carry
        v = ref[pl.ds(g * 16, 16)]
        gm = jnp.max(v)
        gp = jnp.min(jnp.where(v == gm, g * 16 + lanes, jnp.int32(2 ** 30)))
        better = gm > mx
        return jnp.where(better, gm, mx), jnp.where(better, gp, p)

    return lax.fori_loop(0, ngrp, body, (jnp.float32(_NEG), jnp.int32(0)))


def _sc_store1(ref, pos, val):
    plsc.store_scatter(ref, [jnp.full((16,), pos, jnp.int32)],
                       jnp.full((16,), val), mask=_lanes() == 0)


def _read1(ref, pos):
    return ref[pl.ds(pos, 16)][0]


def _nms_stage1(nw, nc):
    def body(prep_hbm, ksco_hbm, kidx_hbm,
             sco_v, cls_v, sx1_v, sy1_v, sx2_v, sy2_v, ar_v,
             cidx_v, csco_v, ccls_v,
             bidx_v, bsco_v, bx1_v, by1_v, bx2_v, by2_v, bar_v,
             k3s_v, k3i_v, ksco_v, kidx_v, dsem):
        w = lax.axis_index("s") * nc + lax.axis_index("c")
        lanes = _lanes()

        cps = [pltpu.async_copy(prep_hbm.at[r], v, dsem)
               for r, v in ((_R_SCO, sco_v), (_R_CLS, cls_v),
                            (_R_SX1, sx1_v), (_R_SY1, sy1_v),
                            (_R_SX2, sx2_v), (_R_SY2, sy2_v),
                            (_R_AR, ar_v))]
        for cp in cps:
            cp.wait()

        base, rem = _NCLS // nw, _NCLS % nw
        lo = jnp.where(w < rem, w * (base + 1),
                       rem * (base + 1) + (w - rem) * base)
        ncls_w = jnp.where(w < rem, base + 1, base)
        flo = lo.astype(jnp.float32)
        fhi = (lo + ncls_w - 1).astype(jnp.float32)

        def scan_body(h, off):
            for u in range(4):
                g = h * 4 + u
                cv = cls_v[pl.ds(g * 16, 16)]
                sv = sco_v[pl.ds(g * 16, 16)]
                mm = (cv >= flo) & (cv <= fhi) & (sv > 0.0)
                plsc.store_compressed(cidx_v.at[pl.ds(off, 16)],
                                      g * 16 + lanes, mask=mm)
                plsc.store_compressed(csco_v.at[pl.ds(off, 16)], sv, mask=mm)
                off = off + jnp.sum(mm.astype(jnp.int32))
            return off

        ncand = lax.fori_loop(0, _NG // 4, scan_body, jnp.int32(0))
        cidx_v[pl.ds(ncand, 16)] = jnp.zeros((16,), jnp.int32)
        csco_v[pl.ds(ncand, 16)] = jnp.full((16,), _NEG, jnp.float32)
        ngrp = (ncand + 15) // 16

        def cgather(g, _):
            ccls_v[pl.ds(g * 16, 16)] = plsc.load_gather(
                cls_v, [cidx_v[pl.ds(g * 16, 16)]])
            return 0

        lax.fori_loop(0, ngrp, cgather, 0)

        kcnt = []
        for t in range(base + 1):
            fc = (lo + t).astype(jnp.float32)

            def subscan(g, off, fc=fc):
                mm = ccls_v[pl.ds(g * 16, 16)] == fc
                plsc.store_compressed(bidx_v.at[pl.ds(off, 16)],
                                      cidx_v[pl.ds(g * 16, 16)], mask=mm)
                plsc.store_compressed(bsco_v.at[pl.ds(off, 16)],
                                      csco_v[pl.ds(g * 16, 16)], mask=mm)
                return off + jnp.sum(mm.astype(jnp.int32))

            nsub = lax.fori_loop(0, ngrp, subscan, jnp.int32(0))
            bidx_v[pl.ds(nsub, 16)] = jnp.zeros((16,), jnp.int32)
            bsco_v[pl.ds(nsub, 16)] = jnp.full((16,), _NEG, jnp.float32)
            sgrp = (nsub + 15) // 16

            def bgather(g, _):
                iv = bidx_v[pl.ds(g * 16, 16)]
                bx1_v[pl.ds(g * 16, 16)] = plsc.load_gather(sx1_v, [iv])
                by1_v[pl.ds(g * 16, 16)] = plsc.load_gather(sy1_v, [iv])
                bx2_v[pl.ds(g * 16, 16)] = plsc.load_gather(sx2_v, [iv])
                by2_v[pl.ds(g * 16, 16)] = plsc.load_gather(sy2_v, [iv])
                bar_v[pl.ds(g * 16, 16)] = plsc.load_gather(ar_v, [iv])
                return 0

            lax.fori_loop(0, sgrp, bgather, 0)

            def cond(carry):
                k, mx, _ = carry
                return (k < _MAXDET) & (mx > 0.0)

            kbase = t * _LCAP

            def gbody(carry, kbase=kbase, sgrp=sgrp):
                k, mx, pos = carry
                gidx = _read1(bidx_v, pos)
                px1 = _read1(bx1_v, pos)
                py1 = _read1(by1_v, pos)
                px2 = _read1(bx2_v, pos)
                py2 = _read1(by2_v, pos)
                par = _read1(bar_v, pos)
                _sc_store1(k3i_v, kbase + k, gidx)
                _sc_store1(k3s_v, kbase + k, mx)

                def supp(g, carry2, px1=px1, py1=py1, px2=px2, py2=py2,
                         par=par, pos=pos):
                    cm, cp = carry2
                    sl = pl.ds(g * 16, 16)
                    ix = jnp.maximum(
                        jnp.minimum(bx2_v[sl], px2)
                        - jnp.maximum(bx1_v[sl], px1), 0.0)
                    iy = jnp.maximum(
                        jnp.minimum(by2_v[sl], py2)
                        - jnp.maximum(by1_v[sl], py1), 0.0)
                    inter = ix * iy
                    iou = inter / (bar_v[sl] + par - inter + 1e-9)
                    kill = (iou > _IOU) | ((g * 16 + lanes) == pos)
                    ns = jnp.where(kill, _NEG, bsco_v[sl])
                    bsco_v[sl] = ns
                    gm = jnp.max(ns)
                    gp = jnp.min(jnp.where(ns == gm, g * 16 + lanes,
                                           jnp.int32(2 ** 30)))
                    better = gm > cm
                    return (jnp.where(better, gm, cm),
                            jnp.where(better, gp, cp))

                m2, p2 = lax.fori_loop(0, sgrp, supp,
                                       (jnp.float32(_NEG), jnp.int32(0)))
                return k + 1, m2, p2

            m0, p0 = _argmax_groups(bsco_v, sgrp)
            kt, _, _ = lax.while_loop(cond, gbody, (jnp.int32(0), m0, p0))
            kcnt.append(kt)

        nls = base + 1

        def hinit(t):
            return jnp.where(kcnt[t] > 0, _read1(k3s_v, t * _LCAP),
                             jnp.float32(_NEG))

        hh = [hinit(t) for t in range(nls)]
        pp = [jnp.int32(0)] * nls

        def mfind(hs):
            mx = hs[0]
            grp = jnp.int32(0)
            for t in range(1, nls):
                better = hs[t] > mx
                mx = jnp.where(better, hs[t], mx)
                grp = jnp.where(better, jnp.int32(t), grp)
            return mx, grp

        def mcond(carry):
            r, mx = carry[0], carry[1]
            return (r < _MAXDET) & (mx > 0.0)

        def mbody(carry):
            r, mx, grp = carry[:3]
            hs = list(carry[3:3 + nls])
            ps = list(carry[3 + nls:3 + 2 * nls])
            hp = ps[0]
            kn = kcnt[0]
            for t in range(1, nls):
                hp = jnp.where(grp == t, ps[t], hp)
                kn = jnp.where(grp == t, kcnt[t], kn)
            gidx = _read1(k3i_v, grp * _LCAP + hp)
            _sc_store1(kidx_v, r, gidx)
            _sc_store1(ksco_v, r, mx)
            hp2 = hp + 1
            nh = jnp.where(
                hp2 < kn,
                _read1(k3s_v, grp * _LCAP + jnp.minimum(hp2, _MAXDET - 1)),
                jnp.float32(_NEG))
            for t in range(nls):
                sel = grp == t
                hs[t] = jnp.where(sel, nh, hs[t])
                ps[t] = jnp.where(sel, hp2, ps[t])
            m2, g2 = mfind(hs)
            return (r + 1, m2, g2, *hs, *ps)

        m0, g0 = mfind(hh)
        fin = lax.while_loop(mcond, mbody, (jnp.int32(0), m0, g0, *hh, *pp))
        kfin = fin[0]

        _sc_store1(kidx_v, _MAXDET, kfin)
        pltpu.sync_copy(ksco_v, ksco_hbm.at[w])
        pltpu.sync_copy(kidx_v, kidx_hbm.at[w])

    return body


def _nms_stage2(nw, nc):
    def body(prep_hbm, ksco_hbm, kidx_hbm, out_hbm,
             sco_l, idx_l, x1_v, y1_v, x2_v, y2_v, cls_v,
             mg_idx, mg_sco, outv, dsem):
        w = lax.axis_index("s") * nc + lax.axis_index("c")
        lanes = _lanes()
        hgrp = nw // 16

        @pl.when(w == 0)
        def _():
            cps = [pltpu.async_copy(src, dst, dsem)
                   for src, dst in ((ksco_hbm, sco_l), (kidx_hbm, idx_l),
                                    (prep_hbm.at[_R_X1], x1_v),
                                    (prep_hbm.at[_R_Y1], y1_v),
                                    (prep_hbm.at[_R_X2], x2_v),
                                    (prep_hbm.at[_R_Y2], y2_v),
                                    (prep_hbm.at[_R_CLS], cls_v))]
            for cp in cps:
                cp.wait()

            def mg_init(g, _):
                mg_idx[pl.ds(g * 16, 16)] = jnp.zeros((16,), jnp.int32)
                return 0

            lax.fori_loop(0, _LCAP // 16, mg_init, 0)

            h, p, cn = [], [], []
            for t in range(hgrp):
                c16 = t * 16 + lanes
                cnt16 = plsc.load_gather(
                    idx_l, [c16, jnp.full((16,), _MAXDET, jnp.int32)])
                h16 = plsc.load_gather(
                    sco_l, [c16, jnp.zeros((16,), jnp.int32)])
                h.append(jnp.where(cnt16 > 0, h16, _NEG))
                p.append(jnp.zeros((16,), jnp.int32))
                cn.append(cnt16)

            def find(hs):
                mx = jnp.max(hs[0])
                grp = jnp.int32(0)
                for t in range(1, hgrp):
                    mt = jnp.max(hs[t])
                    better = mt > mx
                    mx = jnp.where(better, mt, mx)
                    grp = jnp.where(better, jnp.int32(t), grp)
                hsel = hs[0]
                for t in range(1, hgrp):
                    hsel = jnp.where(grp == t, hs[t], hsel)
                lane = jnp.min(jnp.where(hsel == mx, lanes, jnp.int32(16)))
                return mx, grp, lane

            def cond(carry):
                r, mx = carry[0], carry[1]
                return (r < _MAXDET) & (mx > 0.0)

            def mbody(carry):
                r, mx, grp, lane = carry[:4]
                hs = list(carry[4:4 + hgrp])
                ps = list(carry[4 + hgrp:4 + 2 * hgrp])
                lmask = lanes == lane
                psel = ps[0]
                csel = cn[0]
                for t in range(1, hgrp):
                    psel = jnp.where(grp == t, ps[t], psel)
                    csel = jnp.where(grp == t, cn[t], csel)
                hp = jnp.sum(jnp.where(lmask, psel, 0))
                cnl = jnp.sum(jnp.where(lmask, csel, 0))
                cpos = grp * 16 + lane
                cvec = jnp.full((16,), cpos, jnp.int32)
                gidx = plsc.load_gather(
                    idx_l, [cvec, jnp.full((16,), hp, jnp.int32)])[0]
                _sc_store1(mg_idx, r, gidx)
                _sc_store1(mg_sco, r, mx)
                hp2 = hp + 1
                nh_raw = plsc.load_gather(
                    sco_l,
                    [cvec, jnp.full((16,), jnp.minimum(hp2, _MAXDET - 1),
                                    jnp.int32)])[0]
                nhead = jnp.where(hp2 < cnl, nh_raw, jnp.float32(_NEG))
                for t in range(hgrp):
                    sel = (grp == t) & lmask
                    hs[t] = jnp.where(sel, nhead, hs[t])
                    ps[t] = jnp.where(sel, hp2, ps[t])
                m2, g2, l2 = find(hs)
                return (r + 1, m2, g2, l2, *hs, *ps)

            m0, g0, l0 = find(h)
            fin = lax.while_loop(cond, mbody,
                                 (jnp.int32(0), m0, g0, l0, *h, *p))
            rfin = fin[0]

            def asm(g, _):
                rows = g * 16 + lanes
                ok = rows < rfin
                iv = jnp.where(ok, mg_idx[pl.ds(g * 16, 16)], 0)
                flat = rows * 6
                vx1 = jnp.where(ok, plsc.load_gather(x1_v, [iv]), 0.0)
                plsc.store_scatter(outv, [flat], vx1)
                vy1 = jnp.where(ok, plsc.load_gather(y1_v, [iv]), 0.0)
                plsc.store_scatter(outv, [flat + 1], vy1)
                vx2 = jnp.where(ok, plsc.load_gather(x2_v, [iv]), 0.0)
                plsc.store_scatter(outv, [flat + 2], vx2)
                vy2 = jnp.where(ok, plsc.load_gather(y2_v, [iv]), 0.0)
                plsc.store_scatter(outv, [flat + 3], vy2)
                vcf = jnp.where(ok, mg_sco[pl.ds(g * 16, 16)], 0.0)
                plsc.store_scatter(outv, [flat + 4], vcf)
                vcl = jnp.where(ok, plsc.load_gather(cls_v, [iv]), 0.0)
                plsc.store_scatter(outv, [flat + 5], vcl)
                return 0

            lax.fori_loop(0, _LCAP // 16, asm, 0)
            pltpu.sync_copy(outv, out_hbm)

    return body


def kernel(x):
    p = x[0, 0]
    pt = jnp.pad(jnp.transpose(p), ((0, 0), (0, _NPAD - p.shape[0])))
    pt = pt.reshape(_NCLS + 5, _SUB, _LANE)
    prep = pl.pallas_call(
        _prep_body,
        out_shape=jax.ShapeDtypeStruct((16, _SUB, _LANE), jnp.float32),
    )(pt)
    prep = prep.reshape(16, _NPAD)

    info = plsc.get_sparse_core_info()
    nc, ns = info.num_cores, info.num_subcores
    nw = nc * ns
    mesh = plsc.VectorSubcoreMesh(core_axis_name="c", subcore_axis_name="s",
                                  num_cores=nc, num_subcores=ns)
    cp = pltpu.CompilerParams(needs_layout_passes=False)

    f32, i32 = jnp.float32, jnp.int32
    stage1 = functools.partial(
        pl.kernel, _nms_stage1(nw, nc), mesh=mesh, compiler_params=cp,
        out_type=(jax.ShapeDtypeStruct((nw, _LCAP), f32),
                  jax.ShapeDtypeStruct((nw, _LCAP), i32)),
        scratch_types=(
            [pltpu.VMEM((_NPAD,), f32)] * 7
            + [pltpu.VMEM((_CAP,), i32), pltpu.VMEM((_CAP,), f32),
               pltpu.VMEM((_CAP,), f32)]
            + [pltpu.VMEM((_CAP,), i32)]
            + [pltpu.VMEM((_CAP,), f32)] * 6
            + [pltpu.VMEM((944,), f32), pltpu.VMEM((944,), i32)]
            + [pltpu.VMEM((_LCAP,), f32),
               pltpu.VMEM((_LCAP,), i32),
               pltpu.SemaphoreType.DMA]),
    )()
    ksco, kidx = stage1(prep)

    stage2 = functools.partial(
        pl.kernel, _nms_stage2(nw, nc), mesh=mesh, compiler_params=cp,
        out_type=jax.ShapeDtypeStruct((6 * _LCAP,), f32),
        scratch_types=(
            [pltpu.VMEM((nw, _LCAP), f32), pltpu.VMEM((nw, _LCAP), i32)]
            + [pltpu.VMEM((_NPAD,), f32)] * 5
            + [pltpu.VMEM((_LCAP + 16,), i32), pltpu.VMEM((_LCAP + 16,), f32),
               pltpu.VMEM((6 * _LCAP,), f32),
               pltpu.SemaphoreType.DMA]),
    )()
    outflat = stage2(prep, ksco, kidx)
    return outflat[:_MAXDET * 6].reshape(_MAXDET, 6)


def build_and_compile():
    x = jax.ShapeDtypeStruct((1, 1, 5000, 85), jnp.float32)
    jax.jit(kernel).lower(x).compile()
